# baseline (speedup 1.0000x reference)
import functools

import jax
import jax.numpy as jnp
from jax import lax
from jax.experimental import pallas as pl
from jax.experimental.pallas import tpu as pltpu
from jax.experimental.pallas import tpu_sc as plsc

L = 16


@functools.lru_cache(maxsize=None)
def _build(B, F, V, D):
    info = plsc.get_sparse_core_info()
    NC, NS = info.num_cores, info.num_subcores
    NW = NC * NS
    NT = F * D
    assert NT % NW == 0
    TPW = NT // NW
    K = 3
    VC = ((V + K - 1) // K + 127) // 128 * 128
    sizes = [min(VC, V - k * VC) for k in range(K)]
    TAIL = V - (V // 128) * 128
    MAIN_LAST = sizes[K - 1] - TAIL
    NP = TPW * K

    mesh = plsc.VectorSubcoreMesh(core_axis_name="c", subcore_axis_name="s")

    @functools.partial(
        pl.kernel,
        out_type=jax.ShapeDtypeStruct((F, D, B), jnp.float32),
        mesh=mesh,
        scratch_types=[
            pltpu.VMEM((VC,), jnp.float32),
            pltpu.VMEM((VC,), jnp.float32),
            pltpu.VMEM((B,), jnp.int32),
            pltpu.VMEM((B,), jnp.float32),
            pltpu.SemaphoreType.DMA,
            pltpu.SemaphoreType.DMA,
            pltpu.SemaphoreType.DMA,
        ],
        compiler_params=pltpu.CompilerParams(needs_layout_passes=False),
    )
    def emb(xT_hbm, tabT_hbm, tail_hbm, outT_hbm, rowA, rowB, idx_v, out_v,
            semA, semB, sem_out):
        wid = lax.axis_index("s") * NC + lax.axis_index("c")
        rbufs = (rowA, rowB)
        rsems = (semA, semB)

        def fd_of(t):
            fd = wid * TPW + t
            return fd // D, fd % D

        def row_cps(p):
            t, k = divmod(p, K)
            f, d = fd_of(t)
            rbuf, sem = rbufs[p & 1], rsems[p & 1]
            if k < K - 1:
                return [pltpu.make_async_copy(
                    tabT_hbm.at[f, d, pl.ds(k * VC, VC)], rbuf, sem)]
            return [
                pltpu.make_async_copy(
                    tabT_hbm.at[f, d, pl.ds(k * VC, MAIN_LAST)],
                    rbuf.at[pl.ds(0, MAIN_LAST)], sem),
                pltpu.make_async_copy(
                    tail_hbm.at[f, d],
                    rbuf.at[pl.ds(MAIN_LAST, 128)], sem),
            ]

        for cp in row_cps(0):
            cp.start()
        for p in range(NP):
            t, k = divmod(p, K)
            f, d = fd_of(t)
            if p + 1 < NP:
                for cp in row_cps(p + 1):
                    cp.start()
            if k == 0:
                if t == 0:
                    pltpu.sync_copy(xT_hbm.at[f], idx_v)
                else:
                    @pl.when(f != (wid * TPW + t - 1) // D)
                    def _():
                        pltpu.sync_copy(xT_hbm.at[f], idx_v)
            for cp in row_cps(p):
                cp.wait()
            if t > 0 and k == 0:
                pf, pd = fd_of(t - 1)
                pltpu.make_async_copy(out_v, outT_hbm.at[pf, pd], sem_out).wait()
            rbuf = rbufs[p & 1]
            nk = jnp.uint32(sizes[k] - 1)
            if k == 0:
                @plsc.parallel_loop(0, B, step=L, unroll=8)
                def _(b):
                    idx16 = idx_v[pl.ds(b, L)]
                    safe = jnp.minimum(idx16.astype(jnp.uint32), nk)
                    out_v[pl.ds(b, L)] = plsc.load_gather(
                        rbuf, [safe.astype(jnp.int32)])
            elif k < K - 1:
                lo = jnp.int32(k * VC)
                @plsc.parallel_loop(0, B, step=L, unroll=8)
                def _(b):
                    rel = idx_v[pl.ds(b, L)] - lo
                    rel_u = rel.astype(jnp.uint32)
                    safe = jnp.minimum(rel_u, nk).astype(jnp.int32)
                    g = plsc.load_gather(rbuf, [safe])
                    prev = out_v[pl.ds(b, L)]
                    out_v[pl.ds(b, L)] = jnp.where(rel_u <= nk, g, prev)
            else:
                lo = jnp.int32(k * VC)
                jmp_at = jnp.uint32(MAIN_LAST)
                top = jnp.uint32(MAIN_LAST + 127)
                jmp = jnp.uint32(128 - TAIL)
                @plsc.parallel_loop(0, B, step=L, unroll=8)
                def _(b):
                    rel = idx_v[pl.ds(b, L)] - lo
                    rel_u = rel.astype(jnp.uint32)
                    adj = jnp.where(rel_u >= jmp_at, rel_u + jmp, rel_u)
                    safe = jnp.minimum(adj, top).astype(jnp.int32)
                    g = plsc.load_gather(rbuf, [safe])
                    prev = out_v[pl.ds(b, L)]
                    out_v[pl.ds(b, L)] = jnp.where(rel_u <= nk, g, prev)
            if k == K - 1:
                pltpu.async_copy(out_v, outT_hbm.at[f, d], sem_out)
        lf, ld = fd_of(TPW - 1)
        pltpu.make_async_copy(out_v, outT_hbm.at[lf, ld], sem_out).wait()

    return emb


def kernel(x, tables):
    B, F = x.shape
    F2, V, D = tables.shape
    assert F2 == F
    emb = _build(B, F, V, D)
    xT = jnp.swapaxes(x, 0, 1).astype(jnp.int32)
    tabT = jnp.transpose(tables, (0, 2, 1))
    tab_tail = jnp.transpose(tables[:, V - 128:, :], (0, 2, 1))
    outT = emb(xT, tabT, tab_tail)
    return jnp.transpose(outT, (2, 0, 1))


# speedup vs baseline: 1.2641x; 1.2641x over previous
import functools

import jax
import jax.numpy as jnp
from jax import lax
from jax.experimental import pallas as pl
from jax.experimental.pallas import tpu as pltpu
from jax.experimental.pallas import tpu_sc as plsc

L = 16


@functools.lru_cache(maxsize=None)
def _build(B, F, V, D):
    info = plsc.get_sparse_core_info()
    NC, NS = info.num_cores, info.num_subcores
    NW = NC * NS
    NT = F * D
    assert NT % NW == 0
    TPW = NT // NW
    BC = 4096
    while B % BC:
        BC //= 2
    NBC = B // BC

    mesh = plsc.VectorSubcoreMesh(core_axis_name="c", subcore_axis_name="s")

    S1 = (V // 2 + 127) // 128 * 128
    S3 = (V // 128) * 128
    S2 = S3 - S1
    VPAD = S3 + 128

    @functools.partial(
        pl.kernel,
        out_type=jax.ShapeDtypeStruct((F, D, B), jnp.float32),
        mesh=mesh,
        scratch_types=[
            pltpu.VMEM((VPAD,), jnp.float32),
            pltpu.VMEM((B,), jnp.int32),
            pltpu.VMEM((BC,), jnp.float32),
            pltpu.VMEM((BC,), jnp.float32),
            pltpu.SemaphoreType.DMA,
            pltpu.SemaphoreType.DMA,
            pltpu.SemaphoreType.DMA,
        ],
        compiler_params=pltpu.CompilerParams(needs_layout_passes=False),
    )
    def emb(xT_hbm, tabT_hbm, tail_hbm, outT_hbm, row_v, idx_v, outA, outB,
            sem_row, semA, semB):
        wid = lax.axis_index("s") * NC + lax.axis_index("c")
        bufs = (outA, outB)
        sems = (semA, semB)
        for t in range(TPW):
            fd = wid * TPW + t
            f = fd // D
            d = fd % D
            row_cps = [
                pltpu.make_async_copy(
                    tabT_hbm.at[f, d, pl.ds(0, S1)],
                    row_v.at[pl.ds(0, S1)], sem_row),
                pltpu.make_async_copy(
                    tabT_hbm.at[f, d, pl.ds(S1, S2)],
                    row_v.at[pl.ds(S1, S2)], sem_row),
                pltpu.make_async_copy(
                    tail_hbm.at[f, d],
                    row_v.at[pl.ds(S3, 128)], sem_row),
            ]
            for cp in row_cps:
                cp.start()
            if t == 0:
                pltpu.sync_copy(xT_hbm.at[f], idx_v)
            else:
                @pl.when(f != (fd - 1) // D)
                def _():
                    pltpu.sync_copy(xT_hbm.at[f], idx_v)
            for cp in row_cps:
                cp.wait()
            for h in range(NBC):
                c = t * NBC + h
                buf, sem = bufs[c & 1], sems[c & 1]
                dst = outT_hbm.at[f, d, pl.ds(h * BC, BC)]
                if c >= 2:
                    pltpu.make_async_copy(buf, dst, sem).wait()

                @plsc.parallel_loop(0, BC, step=L, unroll=8)
                def _(b):
                    idx16 = idx_v[pl.ds(h * BC + b, L)]
                    buf[pl.ds(b, L)] = plsc.load_gather(row_v, [idx16])

                pltpu.async_copy(buf, dst, sem)
        last = TPW * NBC
        for c in (last - 2, last - 1):
            t, h = c // NBC, c % NBC
            fd = wid * TPW + t
            dst = outT_hbm.at[fd // D, fd % D, pl.ds(h * BC, BC)]
            pltpu.make_async_copy(bufs[c & 1], dst, sems[c & 1]).wait()

    return emb


def kernel(x, tables):
    B, F = x.shape
    F2, V, D = tables.shape
    assert F2 == F
    emb = _build(B, F, V, D)
    xT = jnp.swapaxes(x, 0, 1).astype(jnp.int32)
    tabT = jnp.transpose(tables, (0, 2, 1))
    ntail = V - (V // 128) * 128
    tab_tail = jnp.transpose(
        jnp.pad(tables[:, V - ntail:, :],
                ((0, 0), (0, 128 - ntail), (0, 0)), mode="edge"),
        (0, 2, 1))
    outT = emb(xT, tabT, tab_tail)
    return jnp.transpose(outT, (2, 0, 1))


# speedup vs baseline: 1.2824x; 1.0144x over previous
import functools

import jax
import jax.numpy as jnp
from jax import lax
from jax.experimental import pallas as pl
from jax.experimental.pallas import tpu as pltpu
from jax.experimental.pallas import tpu_sc as plsc

L = 16


@functools.lru_cache(maxsize=None)
def _build(B, F, V, D):
    info = plsc.get_sparse_core_info()
    NC, NS = info.num_cores, info.num_subcores
    NW = NC * NS
    NT = F * D
    assert NT % NW == 0
    TPW = NT // NW
    BC = 4096
    while B % BC:
        BC //= 2
    NBC = B // BC

    mesh = plsc.VectorSubcoreMesh(core_axis_name="c", subcore_axis_name="s")

    @functools.partial(
        pl.kernel,
        out_type=jax.ShapeDtypeStruct((F, D, B), jnp.float32),
        mesh=mesh,
        scratch_types=[
            pltpu.VMEM((V,), jnp.float32),
            pltpu.VMEM((B,), jnp.int32),
            pltpu.VMEM((BC,), jnp.float32),
            pltpu.VMEM((BC,), jnp.float32),
            pltpu.SemaphoreType.DMA,
            pltpu.SemaphoreType.DMA,
        ],
        compiler_params=pltpu.CompilerParams(needs_layout_passes=False),
    )
    def emb(xT_hbm, tabT_hbm, outT_hbm, row_v, idx_v, outA, outB, semA, semB):
        wid = lax.axis_index("s") * NC + lax.axis_index("c")
        bufs = (outA, outB)
        sems = (semA, semB)
        for t in range(TPW):
            fd = wid * TPW + t
            f = fd // D
            d = fd % D
            pltpu.sync_copy(tabT_hbm.at[f, d], row_v)
            if t == 0:
                pltpu.sync_copy(xT_hbm.at[f], idx_v)
            else:
                @pl.when(f != (fd - 1) // D)
                def _():
                    pltpu.sync_copy(xT_hbm.at[f], idx_v)
            for h in range(NBC):
                c = t * NBC + h
                buf, sem = bufs[c & 1], sems[c & 1]
                dst = outT_hbm.at[f, d, pl.ds(h * BC, BC)]
                if c >= 2:
                    pltpu.make_async_copy(buf, dst, sem).wait()

                @plsc.parallel_loop(0, BC, step=L, unroll=8)
                def _(b):
                    idx16 = idx_v[pl.ds(h * BC + b, L)]
                    buf[pl.ds(b, L)] = plsc.load_gather(row_v, [idx16])

                pltpu.async_copy(buf, dst, sem)
        last = TPW * NBC
        for c in (last - 2, last - 1):
            t, h = c // NBC, c % NBC
            fd = wid * TPW + t
            dst = outT_hbm.at[fd // D, fd % D, pl.ds(h * BC, BC)]
            pltpu.make_async_copy(bufs[c & 1], dst, sems[c & 1]).wait()

    return emb


def kernel(x, tables):
    B, F = x.shape
    F2, V, D = tables.shape
    assert F2 == F
    emb = _build(B, F, V, D)
    xT = jnp.swapaxes(x, 0, 1).astype(jnp.int32)
    tabT = jnp.transpose(tables, (0, 2, 1))
    outT = emb(xT, tabT)
    return jnp.transpose(outT, (2, 0, 1))
